# baseline (speedup 1.0000x reference)
import functools

import jax
import jax.numpy as jnp
from jax import lax
from jax.experimental import pallas as pl
from jax.experimental.pallas import tpu as pltpu
from jax.experimental.pallas import tpu_sc as plsc

N = 10000
E = 320000
F_IN = 128
HID = 64
NCLS = 32
T = 4

NC = 2
NS = 16
NW = NC * NS
CH = 80
NR = E // CH
RW = NR // NW
NPT = N // NS

_BLK = 2000
_GRID = N // _BLK


def _proj_body(x_ref, w_ref, y_ref, s_ref, dout):
    y = jnp.dot(x_ref[...], w_ref[...], preferred_element_type=jnp.float32,
                precision=lax.Precision.HIGHEST)
    y_ref[...] = y
    acc = y[:, 0:dout]
    for t in range(1, T):
        acc = acc + y[:, t * dout:(t + 1) * dout]
    s_ref[...] = acc


def _proj(x, wc, dout):
    din = x.shape[1]
    return pl.pallas_call(
        functools.partial(_proj_body, dout=dout),
        grid=(_GRID,),
        in_specs=[
            pl.BlockSpec((_BLK, din), lambda i: (i, 0)),
            pl.BlockSpec((din, T * dout), lambda i: (0, 0)),
        ],
        out_specs=[
            pl.BlockSpec((_BLK, T * dout), lambda i: (i, 0)),
            pl.BlockSpec((_BLK, dout), lambda i: (i, 0)),
        ],
        out_shape=[
            jax.ShapeDtypeStruct((N, T * dout), jnp.float32),
            jax.ShapeDtypeStruct((N, dout), jnp.float32),
        ],
    )(x, wc)


def _mid_body(s_ref, p_ref, b_ref, w_ref, y_ref, s2_ref, dout):
    pp = p_ref[:, 0:HID] + p_ref[:, HID:2 * HID]
    h = jnp.maximum(s_ref[...] + pp + b_ref[...], 0.0)
    y = jnp.dot(h, w_ref[...], preferred_element_type=jnp.float32,
                precision=lax.Precision.HIGHEST)
    y_ref[...] = y
    acc = y[:, 0:dout]
    for t in range(1, T):
        acc = acc + y[:, t * dout:(t + 1) * dout]
    s2_ref[...] = acc


def _mid(s1, p, b1, wc2):
    din, dout = HID, NCLS
    return pl.pallas_call(
        functools.partial(_mid_body, dout=dout),
        grid=(_GRID,),
        in_specs=[
            pl.BlockSpec((_BLK, din), lambda i: (i, 0)),
            pl.BlockSpec((_BLK, NC * HID), lambda i: (i, 0)),
            pl.BlockSpec((1, din), lambda i: (0, 0)),
            pl.BlockSpec((din, T * dout), lambda i: (0, 0)),
        ],
        out_specs=[
            pl.BlockSpec((_BLK, T * dout), lambda i: (i, 0)),
            pl.BlockSpec((_BLK, dout), lambda i: (i, 0)),
        ],
        out_shape=[
            jax.ShapeDtypeStruct((N, T * dout), jnp.float32),
            jax.ShapeDtypeStruct((N, dout), jnp.float32),
        ],
    )(s1, p, b1, wc2)


def _fin_body(s_ref, q_ref, b_ref, wss_ref, bss_ref, out_ref, tgt_ref):
    qq = q_ref[:, 0:NCLS] + q_ref[:, NCLS:2 * NCLS]
    o = jnp.maximum(s_ref[...] + qq + b_ref[...], 0.0)
    out_ref[...] = o
    tgt_ref[...] = (
        jnp.dot(o, wss_ref[...], preferred_element_type=jnp.float32,
                precision=lax.Precision.HIGHEST)
        + bss_ref[...]
    )


def _fin(s2, q, b2, wss, bss):
    return pl.pallas_call(
        _fin_body,
        grid=(_GRID,),
        in_specs=[
            pl.BlockSpec((_BLK, NCLS), lambda i: (i, 0)),
            pl.BlockSpec((_BLK, 128), lambda i: (i, 0)),
            pl.BlockSpec((1, NCLS), lambda i: (0, 0)),
            pl.BlockSpec((NCLS, F_IN), lambda i: (0, 0)),
            pl.BlockSpec((1, F_IN), lambda i: (0, 0)),
        ],
        out_specs=[
            pl.BlockSpec((_BLK, NCLS), lambda i: (i, 0)),
            pl.BlockSpec((_BLK, F_IN), lambda i: (i, 0)),
        ],
        out_shape=[
            jax.ShapeDtypeStruct((N, NCLS), jnp.float32),
            jax.ShapeDtypeStruct((N, F_IN), jnp.float32),
        ],
    )(s2, q, b2, wss, bss)


G = 5
NG = RW // G


def _sc_body(src_hbm, tim_hbm, dst_hbm, tab_hbm, zro_hbm, out_hbm,
             idx_v, tring, dring, row_v, acc_sh, gsem, ssem, hsem,
             tsem, dsem, *, d, sets, ahead, wide):
    c = lax.axis_index("c")
    s = lax.axis_index("s")
    wid = s * NC + c
    base = wid * RW
    cps = [
        (src_hbm.at[pl.ds(base, RW)], idx_v),
        (zro_hbm.at[pl.ds(s * NPT, NPT)], acc_sh.at[pl.ds(s * NPT, NPT)]),
    ]
    for a, b in cps:
        pltpu.async_copy(a, b, hsem)

    def _fire_t(g):
        pltpu.async_copy(tim_hbm.at[pl.ds(base + g * G, G)],
                         tring.at[lax.rem(g, 2)], tsem.at[lax.rem(g, 2)])

    def _drain_t(g):
        pltpu.make_async_copy(tim_hbm.at[pl.ds(base + g * G, G)],
                              tring.at[lax.rem(g, 2)],
                              tsem.at[lax.rem(g, 2)]).wait()

    def _fire_d(g):
        pltpu.async_copy(dst_hbm.at[pl.ds(base + g * G, G)],
                         dring.at[lax.rem(g, 3)], dsem.at[lax.rem(g, 3)])

    def _drain_d(g):
        pltpu.make_async_copy(dst_hbm.at[pl.ds(base + g * G, G)],
                              dring.at[lax.rem(g, 3)],
                              dsem.at[lax.rem(g, 3)]).wait()

    _fire_t(0)
    _fire_t(1)
    _fire_d(0)
    _fire_d(1)
    for a, b in cps:
        pltpu.make_async_copy(a, b, hsem).wait()

    def _ix_grp(g):
        _drain_t(g)
        ts = lax.rem(g, 2)
        for j in range(G):
            for k in range(CH // 16):
                sl = pl.ds(k * 16, 16)
                idx_v[g * G + j, sl] = (idx_v[g * G + j, sl] * T
                                        + tring[ts, j, sl])

    plsc.subcore_barrier()

    def _fire_g(g, m, sem):
        for i in range(G):
            pltpu.async_copy(tab_hbm.at[idx_v.at[g * G + i]],
                             row_v.at[m, i], sem)

    def _drain_g(g, m, sem):
        for i in range(G):
            pltpu.make_async_copy(tab_hbm.at[idx_v.at[g * G + i]],
                                  row_v.at[m, i], sem).wait()

    def _fire_s(g, m, sem):
        ds_ = lax.rem(g, 3)
        for i in range(G):
            pltpu.async_copy(row_v.at[m, i],
                             acc_sh.at[dring.at[ds_, i]], sem, add=True)

    def _drain_s(g, m, sem):
        ds_ = lax.rem(g, 3)
        for i in range(G):
            pltpu.make_async_copy(row_v.at[m, i],
                                  acc_sh.at[dring.at[ds_, i]], sem).wait()

    for a in range(ahead):
        _ix_grp(a)
        if a + ahead < NG:
            _fire_t(a + ahead)
        _fire_g(a, a, gsem.at[a % 4])

    def _grp(g, carry):
        m = lax.rem(g, sets)
        ma = lax.rem(g + ahead, sets)
        e = lax.rem(g, 2)

        @pl.when(g + ahead < NG)
        def _():
            _ix_grp(g + ahead)

            @pl.when(g + ahead + 2 < NG)
            def _():
                _fire_t(g + ahead + 2)

        _drain_g(g, m, gsem.at[lax.rem(g, 4)])
        _drain_d(g)
        _fire_s(g, m, ssem.at[e])

        @pl.when(g >= 1)
        def _():
            _drain_s(g - 1, lax.rem(g - 1, sets), ssem.at[1 - e])

        @pl.when(g + 2 < NG)
        def _():
            _fire_d(g + 2)

        @pl.when(g + ahead < NG)
        def _():
            _fire_g(g + ahead, ma, gsem.at[lax.rem(g + ahead, 4)])

        return carry

    lax.fori_loop(0, NG, _grp, 0)
    _drain_s(NG - 1, (NG - 1) % sets, ssem.at[(NG - 1) % 2])
    plsc.subcore_barrier()
    if wide:
        pltpu.sync_copy(acc_sh.at[pl.ds(s * NPT, NPT)],
                        out_hbm.at[pl.ds(s * NPT, NPT), pl.ds(c * d, d)])
    else:
        pltpu.sync_copy(acc_sh.at[pl.ds(s * NPT, NPT)],
                        out_hbm.at[c, pl.ds(s * NPT, NPT)])


def _sc_scatter(d, sets, ahead, wide=False):
    mesh = plsc.VectorSubcoreMesh(
        core_axis_name="c", subcore_axis_name="s",
        num_cores=NC, num_subcores=NS)
    oshape = (N, 128) if wide else (NC, N, d)
    return pl.kernel(
        functools.partial(_sc_body, d=d, sets=sets, ahead=ahead, wide=wide),
        out_type=jax.ShapeDtypeStruct(oshape, jnp.float32),
        mesh=mesh,
        compiler_params=pltpu.CompilerParams(use_tc_tiling_on_sc=False),
        scratch_types=[
            pltpu.VMEM((RW, CH), jnp.int32),
            pltpu.VMEM((2, G, CH), jnp.int32),
            pltpu.VMEM((3, G, CH), jnp.int32),
            pltpu.VMEM((sets, G, CH, d), jnp.float32),
            pltpu.VMEM_SHARED((N, d), jnp.float32),
            pltpu.SemaphoreType.DMA((4,)),
            pltpu.SemaphoreType.DMA((2,)),
            pltpu.SemaphoreType.DMA,
            pltpu.SemaphoreType.DMA((2,)),
            pltpu.SemaphoreType.DMA((3,)),
        ],
    )


def kernel(x, edge_index, time_index, W1, b1, W2, b2, W_ss, b_ss):
    src = edge_index[0].reshape(NR, CH)
    dst = edge_index[1].reshape(NR, CH)
    tim = time_index.reshape(NR, CH)

    wc1 = W1.reshape(T, F_IN, HID).transpose(1, 0, 2).reshape(F_IN, T * HID)
    wc2 = W2.reshape(T, HID, NCLS).transpose(1, 0, 2).reshape(HID, T * NCLS)

    y1, s1 = _proj(x, wc1, HID)
    tab1 = y1.reshape(N * T, HID)
    p = _sc_scatter(HID, 3, 2, wide=True)(src, tim, dst, tab1,
                                          jnp.zeros((N, HID), jnp.float32))
    y2, s2 = _mid(s1, p, b1.reshape(1, HID), wc2)
    tab2 = y2.reshape(N * T, NCLS)
    q = _sc_scatter(NCLS, 4, 2, wide=True)(src, tim, dst, tab2,
                                           jnp.zeros((N, NCLS), jnp.float32))
    out, tgt = _fin(s2, q, b2.reshape(1, NCLS), W_ss,
                    b_ss.reshape(1, F_IN))
    return (out, tgt)
